# baseline (speedup 1.0000x reference)
import functools

import jax
import jax.numpy as jnp
from jax import lax
from jax.experimental import pallas as pl
from jax.experimental.pallas import tpu as pltpu, tpu_sc as plsc

_B = 16384
_D = 64
_V = 1000001
_NC = 2
_NS = 16
_NW = _NC * _NS
_NTC = (_V + 127) // 128
_RANGE = 245
_WTC = 2
_LANES = _WTC * 128
_NBUF = 5
_NPASS = 125
_NOUTER = _NPASS // _NBUF
_NSUB = 8
_LCHUNK = 2048
_OUTR = _B + _NW


def _make_scan():
  mesh = plsc.VectorSubcoreMesh(core_axis_name="c", subcore_axis_name="s")

  @functools.partial(
      pl.kernel,
      out_type=jax.ShapeDtypeStruct((_OUTR, 128), jnp.float32),
      mesh=mesh,
      scratch_types=[
          pltpu.VMEM((_LCHUNK,), jnp.int32),
          pltpu.VMEM((_B + 16,), jnp.int32),
          pltpu.VMEM((_B + 16,), jnp.int32),
          pltpu.VMEM((_NBUF, _D, _LANES), jnp.float32),
          pltpu.VMEM((2, 32, 128), jnp.float32),
          pltpu.VMEM((2, 32), jnp.int32),
          pltpu.SMEM((_NSUB,), jnp.int32),
          pltpu.SMEM((_NSUB,), jnp.int32),
      ] + [pltpu.SemaphoreType.DMA] * (_NBUF + 1),
      compiler_params=pltpu.CompilerParams(needs_layout_passes=False),
  )
  def scan(labels_hbm, tt_hbm, out_hbm, labbuf, ml, wl, win, stage, pend,
           seg_s, seg_n, *sems):
    wsems, semo = sems[:_NBUF], sems[_NBUF]
    wid = lax.axis_index("s") * _NC + lax.axis_index("c")
    iota = lax.iota(jnp.int32, 16)
    lo_tc = wid * _RANGE
    hi_tc = jnp.minimum(lo_tc + _RANGE, _NTC)
    lo_lane = lo_tc * 128

    def lane_off(k):
      c0 = jnp.minimum(lo_tc + k * _WTC, _NTC - _WTC)
      return c0 * 128

    def fetch(k, buf, sem):
      off = lane_off(k)
      pltpu.async_copy(
          tt_hbm.at[pl.ds(0, 32), pl.ds(off, _LANES)], buf.at[pl.ds(0, 32)],
          sem)
      pltpu.async_copy(
          tt_hbm.at[pl.ds(32, 32), pl.ds(off, _LANES)], buf.at[pl.ds(32, 32)],
          sem)

    for kk in range(_NBUF):
      fetch(kk, win.at[kk], wsems[kk])

    def preo(o, cnt):
      pltpu.sync_copy(labels_hbm.at[pl.ds(o * _LCHUNK, _LCHUNK)], labbuf)

      def pre(c, cnt):
        lv = labbuf[pl.ds(c * 16, 16)]
        tc = lax.shift_right_logical(lv, 7)
        m = (tc >= lo_tc) & (tc < hi_tc)
        cum = plsc.cumsum(jnp.where(m, 1, 0))
        packed = lax.shift_left(lv - lo_lane, 14) | (o * _LCHUNK + c * 16 + iota)
        plsc.store_scatter(wl, [cnt + cum - 1], packed, mask=m)
        return cnt + cum[15]

      return lax.fori_loop(0, _LCHUNK // 16, pre, cnt)

    cnt = lax.fori_loop(0, _B // _LCHUNK, preo, 0)
    nch = (cnt + 15) // 16

    def mksub(r, off):
      seg_s[r] = off

      def srt(j, o):
        jv = j * 16 + iota
        pk = wl[pl.ds(j * 16, 16)]
        sr = lax.shift_right_logical(pk, 26)
        m = (jv < cnt) & (sr == r)
        cum = plsc.cumsum(jnp.where(m, 1, 0))
        plsc.store_scatter(ml, [o + cum - 1], pk, mask=m)
        return o + cum[15]

      off2 = lax.fori_loop(0, nch, srt, off)
      seg_n[r] = off2 - seg_s[r]
      return off2

    off = 0
    for r in range(_NSUB):
      off = mksub(r, off)

    def window_pass(k, buf, sem, p):
      pltpu.make_async_copy(
          tt_hbm.at[:, pl.ds(0, _LANES)], buf, sem).wait()
      c0rel = jnp.minimum(lo_tc + k * _WTC, _NTC - _WTC) - lo_tc
      r = k >> 4
      s0 = seg_s[r]
      sn = seg_n[r]
      swch = (sn + 15) // 16

      def wfil(j, wcnt):
        jloc = j * 16 + iota
        pk = ml[pl.ds(s0 + j * 16, 16)]
        tcr = lax.shift_right_logical(pk, 21)
        m = (jloc < sn) & (tcr >= c0rel) & (tcr < c0rel + _WTC)
        cum = plsc.cumsum(jnp.where(m, 1, 0))
        plsc.store_scatter(wl, [wcnt + cum - 1], pk, mask=m)
        return wcnt + cum[15]

      wcnt = lax.fori_loop(0, swch, wfil, 0)

      def ext(i, p):
        pk = plsc.load_gather(wl, [jnp.broadcast_to(i, (16,))])
        lane = lax.shift_right_logical(pk, 14) - c0rel * 128
        b_s = pk & 16383
        f = (p // 32) % 2
        slot = p % 32
        for ch in range(_D // 16):
          d_idx = ch * 16 + iota
          vals = plsc.load_gather(buf, [d_idx, lane])
          stage[f, slot, pl.ds(ch * 16, 16)] = vals
        plsc.store_scatter(
            pend.at[f], [jnp.broadcast_to(slot, (16,))], b_s, mask=iota == 0)
        p1 = p + 1

        @pl.when(p1 % 32 == 0)
        def _flush():
          pltpu.async_copy(stage.at[f], out_hbm.at[pend.at[f]], semo).wait()

        return p1

      p = lax.fori_loop(0, wcnt, ext, p)
      nk = k + _NBUF

      @pl.when(nk < _NPASS)
      def _refetch():
        fetch(nk, buf, sem)

      return p

    def outer(t, st):
      for kk in range(_NBUF):
        st = window_pass(_NBUF * t + kk, win.at[kk], wsems[kk], st)
      return st

    p = lax.fori_loop(0, _NOUTER, outer, 0)

    f = (p // 32) % 2
    rem = p % 32
    trash = _B + ((jnp.broadcast_to(wid, (16,)) + iota) % _NW)
    for h in range(2):
      plsc.store_scatter(
          pend.at[f], [iota + 16 * h], trash, mask=(iota + 16 * h) >= rem)
    pltpu.async_copy(stage.at[f], out_hbm.at[pend.at[f]], semo).wait()

  return scan


_scan = _make_scan()


@jax.jit
def kernel(labels, embedding_table):
  out_wide = _scan(labels.astype(jnp.int32), embedding_table.T)
  return out_wide[:_B, :_D]


# speedup vs baseline: 1.0050x; 1.0050x over previous
import functools

import jax
import jax.numpy as jnp
from jax import lax
from jax.experimental import pallas as pl
from jax.experimental.pallas import tpu as pltpu, tpu_sc as plsc

_B = 16384
_D = 64
_V = 1000001
_NC = 2
_NS = 16
_NW = _NC * _NS
_NTC = (_V + 127) // 128
_RANGE = 245
_WTC = 2
_LANES = _WTC * 128
_NWIN = 123
_NOUTER = 31
_NSUB = 8
_OUTR = _B + _NW


def _make_scan():
  mesh = plsc.VectorSubcoreMesh(core_axis_name="c", subcore_axis_name="s")

  @functools.partial(
      pl.kernel,
      out_type=jax.ShapeDtypeStruct((_OUTR, 128), jnp.float32),
      mesh=mesh,
      scratch_types=[
          pltpu.VMEM((_B + 16,), jnp.int32),
          pltpu.VMEM((_B + 16,), jnp.int32),
          pltpu.VMEM((_B + 16,), jnp.int32),
          pltpu.VMEM((4, _D, _LANES), jnp.float32),
          pltpu.VMEM((2, 32, 128), jnp.float32),
          pltpu.VMEM((2, 32), jnp.int32),
          pltpu.SMEM((_NSUB,), jnp.int32),
          pltpu.SMEM((_NSUB,), jnp.int32),
          pltpu.SemaphoreType.DMA,
          pltpu.SemaphoreType.DMA,
          pltpu.SemaphoreType.DMA,
          pltpu.SemaphoreType.DMA,
          pltpu.SemaphoreType.DMA,
      ],
      compiler_params=pltpu.CompilerParams(needs_layout_passes=False),
  )
  def scan(labels_hbm, tt_hbm, out_hbm, lab, sj, wj, win, stage, pend,
           seg_s, seg_n, sem0, sem1, sem2, sem3, semo):
    wid = lax.axis_index("s") * _NC + lax.axis_index("c")
    iota = lax.iota(jnp.int32, 16)
    lo_tc = wid * _RANGE
    hi_tc = jnp.minimum(lo_tc + _RANGE, _NTC)

    def lane_off(k):
      c0 = jnp.minimum(lo_tc + k * _WTC, _NTC - _WTC)
      return c0 * 128

    def fetch(k, buf, sem):
      off = lane_off(k)
      pltpu.async_copy(
          tt_hbm.at[pl.ds(0, 32), pl.ds(off, _LANES)], buf.at[pl.ds(0, 32)],
          sem)
      pltpu.async_copy(
          tt_hbm.at[pl.ds(32, 32), pl.ds(off, _LANES)], buf.at[pl.ds(32, 32)],
          sem)

    wsems = (sem0, sem1, sem2, sem3)
    for kk in range(4):
      fetch(kk, win.at[kk], wsems[kk])

    pltpu.sync_copy(labels_hbm, lab.at[pl.ds(0, _B)])

    def pre(c, cnt):
      lv = lab[pl.ds(c * 16, 16)]
      tc = lax.shift_right_logical(lv, 7)
      m = (tc >= lo_tc) & (tc < hi_tc)
      cum = plsc.cumsum(jnp.where(m, 1, 0))
      plsc.store_scatter(wj, [cnt + cum - 1], c * 16 + iota, mask=m)
      return cnt + cum[15]

    cnt = lax.fori_loop(0, _B // 16, pre, 0)
    nch = (cnt + 15) // 16

    def mksub(r, off):
      seg_s[r] = off

      def srt(j, o):
        jv = j * 16 + iota
        valid = jv < cnt
        bs = wj[pl.ds(j * 16, 16)]
        lv = plsc.load_gather(lab, [bs], mask=valid)
        sr = lax.shift_right_logical(lax.shift_right_logical(lv, 7) - lo_tc, 5)
        m = valid & (sr == r)
        cum = plsc.cumsum(jnp.where(m, 1, 0))
        plsc.store_scatter(sj, [o + cum - 1], bs, mask=m)
        return o + cum[15]

      off2 = lax.fori_loop(0, nch, srt, off)
      seg_n[r] = off2 - seg_s[r]
      return off2

    off = 0
    for r in range(_NSUB):
      off = mksub(r, off)

    def window_pass(k, buf, sem, p):
      pltpu.make_async_copy(
          tt_hbm.at[:, pl.ds(0, _LANES)], buf, sem).wait()
      c0n = jnp.minimum(lo_tc + k * _WTC, _NTC - _WTC)
      base_lane = c0n * 128
      r = k >> 4
      s0 = seg_s[r]
      sn = seg_n[r]
      swch = (sn + 15) // 16

      def wfil(j, wcnt):
        jloc = j * 16 + iota
        valid = jloc < sn
        bs = sj[pl.ds(s0 + j * 16, 16)]
        lv = plsc.load_gather(lab, [bs], mask=valid)
        tc = lax.shift_right_logical(lv, 7)
        m = valid & (tc >= c0n) & (tc < c0n + _WTC)
        cum = plsc.cumsum(jnp.where(m, 1, 0))
        plsc.store_scatter(wj, [wcnt + cum - 1], bs, mask=m)
        return wcnt + cum[15]

      wcnt = lax.fori_loop(0, swch, wfil, 0)

      def ext(i, p):
        b_s = plsc.load_gather(wj, [jnp.broadcast_to(i, (16,))])
        l_s = plsc.load_gather(lab, [b_s])
        lane = l_s - base_lane
        f = (p // 32) % 2
        slot = p % 32
        for ch in range(_D // 16):
          d_idx = ch * 16 + iota
          vals = plsc.load_gather(buf, [d_idx, lane])
          stage[f, slot, pl.ds(ch * 16, 16)] = vals
        plsc.store_scatter(
            pend.at[f], [jnp.broadcast_to(slot, (16,))], b_s, mask=iota == 0)
        p1 = p + 1

        @pl.when(p1 % 32 == 0)
        def _flush():
          pltpu.async_copy(stage.at[f], out_hbm.at[pend.at[f]], semo).wait()

        return p1

      p = lax.fori_loop(0, wcnt, ext, p)
      nk = k + 4

      @pl.when(nk < _NOUTER * 4)
      def _refetch():
        fetch(nk, buf, sem)

      return p

    def outer(t, st):
      for kk in range(4):
        st = window_pass(4 * t + kk, win.at[kk], wsems[kk], st)
      return st

    p = lax.fori_loop(0, _NOUTER, outer, 0)

    f = (p // 32) % 2
    rem = p % 32
    trash = _B + ((jnp.broadcast_to(wid, (16,)) + iota) % _NW)
    for h in range(2):
      plsc.store_scatter(
          pend.at[f], [iota + 16 * h], trash, mask=(iota + 16 * h) >= rem)
    pltpu.async_copy(stage.at[f], out_hbm.at[pend.at[f]], semo).wait()

  return scan


_scan = _make_scan()


@jax.jit
def kernel(labels, embedding_table):
  out_wide = _scan(labels.astype(jnp.int32), embedding_table.T)
  return out_wide[:_B, :_D]
